# Initial kernel scaffold; baseline (speedup 1.0000x reference)
import jax
import jax.numpy as jnp
from jax.experimental import pallas as pl


def kernel(x, W):
    raise NotImplementedError("write your pallas kernel here")


# speedup vs baseline: 4.1549x; 4.1549x over previous
import functools

import jax
import jax.numpy as jnp
from jax import lax
from jax.experimental import pallas as pl
from jax.experimental.pallas import tpu as pltpu
from jax.experimental.pallas import tpu_sc as plsc

D = 64
GRP = 128
CHUNK = 4
SUPER = 32
NC, NS = 2, 16
NW = NC * NS


def _body(idx_hbm, table_hbm, out_hbm, idx_v, rows_v, sem, *, groups):
    groups_per_w = groups // NW
    wid = lax.axis_index("s") * NC + lax.axis_index("c")
    g0 = wid * groups_per_w

    def outer(s, _):
        sg0 = g0 + s * SUPER
        pltpu.sync_copy(idx_hbm.at[pl.ds(sg0, SUPER)], idx_v)

        def inner(j, _):
            cps = [
                pltpu.async_copy(
                    table_hbm.at[idx_v.at[j * CHUNK + k]],
                    rows_v.at[pl.ds(k * GRP, GRP)],
                    sem,
                )
                for k in range(CHUNK)
            ]
            for cp in cps:
                cp.wait()
            row0 = (sg0 + j * CHUNK) * GRP
            pltpu.sync_copy(rows_v, out_hbm.at[pl.ds(row0, CHUNK * GRP)])
            return 0

        lax.fori_loop(0, SUPER // CHUNK, inner, 0)
        return 0

    lax.fori_loop(0, groups_per_w // SUPER, outer, 0)


@functools.partial(jax.jit, static_argnames=("groups",))
def _gather(idx, table, *, groups):
    body = functools.partial(_body, groups=groups)
    return pl.kernel(
        body,
        out_type=jax.ShapeDtypeStruct((groups * GRP, D), jnp.float32),
        mesh=plsc.VectorSubcoreMesh(core_axis_name="c", subcore_axis_name="s"),
        scratch_types=[
            pltpu.VMEM((SUPER, GRP), jnp.int32),
            pltpu.VMEM((CHUNK * GRP, D), jnp.float32),
            pltpu.SemaphoreType.DMA,
        ],
        compiler_params=pltpu.CompilerParams(use_tc_tiling_on_sc=False),
    )(idx, table)


def kernel(x, W):
    b, h = x.shape
    groups = (b * h) // GRP
    idx = x.reshape(groups, GRP).astype(jnp.int32)
    out = _gather(idx, W, groups=groups)
    return out.reshape(b, h, D)
